# baseline (speedup 1.0000x reference)
import functools
import jax
import jax.numpy as jnp
from jax import lax
from jax.experimental import pallas as pl
from jax.experimental.pallas import tpu as pltpu
from jax.experimental.pallas import tpu_sc as plsc

_K = 64
_N = 32768
_ROWS = 128
_L = 16
_INT_MIN32 = -2147483648


def _splat(s):
    return jnp.full((_L,), s, jnp.int32)


def _bitval(b):
    return jnp.int32(_INT_MIN32) if b == 31 else jnp.int32(1 << b)


def _popcount(mask):
    return plsc.all_reduce_population_count(mask)


def _lane_total(acc, nbits=12):
    tot = _splat(0)
    for b in range(nbits):
        pc = _popcount(((acc >> b) & 1) == 1)
        tot = tot + (pc << b)
    return tot


def _vreg_kth(vec, k):
    prefix = _splat(_INT_MIN32)
    for b in range(31, -1, -1):
        tr = prefix ^ _bitval(b)
        c = _popcount(vec >= tr)
        prefix = jnp.where(c >= k, tr, prefix)
    return prefix


def _sc_body(x_hbm, out_hbm, row_v, key_v, sem):
    wid = lax.axis_index("s") * 2 + lax.axis_index("c")
    rows_per_w = _ROWS // 32

    for rr in range(rows_per_w):
        row = wid * rows_per_w + rr
        pltpu.sync_copy(x_hbm.at[row], row_v)

        def mx_step(j, accs):
            base = j * 64
            out = []
            for g in range(4):
                sl = pl.ds(base + g * _L, _L)
                raw = lax.bitcast_convert_type(row_v[sl], jnp.int32)
                k = jnp.where(raw >= 0, raw, raw ^ jnp.int32(0x7FFFFFFF))
                key_v[sl] = k
                out.append(jnp.maximum(accs[g], k))
            return tuple(out)

        a = lax.fori_loop(0, _N // 64, mx_step,
                          tuple(_splat(_INT_MIN32) for _ in range(4)))
        mall = jnp.maximum(jnp.maximum(a[0], a[1]), jnp.maximum(a[2], a[3]))
        mmin = jnp.minimum(jnp.minimum(a[0], a[1]), jnp.minimum(a[2], a[3]))
        u = _vreg_kth(mall, 1)
        lo0 = _vreg_kth(mmin, _L)

        def count_ge(tsplat):
            def step(j, acc):
                base = j * 64
                s = acc
                for g in range(4):
                    k = key_v[pl.ds(base + g * _L, _L)]
                    s = s + (k >= tsplat).astype(jnp.int32)
                return s

            acc = lax.fori_loop(0, _N // 64, step, _splat(0))
            return _lane_total(acc)

        cl0 = count_ge(lo0)
        hi0 = u + 1
        ch0 = _splat(0)

        def active(lo, hi, cl):
            return (cl > _K) & (hi - 1 > lo)

        def loop_cond(s):
            it, lo, hi, cl, ch = s
            return jnp.any(active(lo, hi, cl))

        def loop_body(s):
            it, lo, hi, cl, ch = s
            wf = hi.astype(jnp.float32) - lo.astype(jnp.float32)
            frac = (cl - _K).astype(jnp.float32) / jnp.maximum(
                (cl - ch).astype(jnp.float32), 1.0)
            stepf = jnp.clip(wf * frac, 1.0, jnp.maximum(wf - 1.0, 1.0))
            mid_i = lo + stepf.astype(jnp.int32)
            mid_b = (lo & hi) + ((lo ^ hi) >> 1)
            mid = jnp.where(it % 2 == 0, mid_i, mid_b)
            mid = jnp.maximum(lo + 1, jnp.minimum(mid, hi - 1))
            c = count_ge(mid)
            up = c >= _K
            lo = jnp.where(up, mid, lo)
            cl = jnp.where(up, c, cl)
            hi = jnp.where(up, hi, mid)
            ch = jnp.where(up, ch, c)
            return it + 1, lo, hi, cl, ch

        _, lo, hi, cl, ch = lax.while_loop(
            loop_cond, loop_body, (jnp.int32(0), lo0, hi0, cl0, ch0))
        t = lo

        def tie_cutoff():
            cnt_gt = count_ge(t + 1)
            need = _K - cnt_gt

            def csel(trs):
                def step(j, acc):
                    base = j * 64
                    s = acc
                    for g in range(4):
                        off = base + g * _L
                        k = key_v[pl.ds(off, _L)]
                        idx = lax.iota(jnp.int32, _L) + off
                        s = s + ((k == t) & (idx < trs)).astype(jnp.int32)
                    return s

                acc = lax.fori_loop(0, _N // 64, step, _splat(0))
                return _lane_total(acc)

            ipref = _splat(0)
            for b in range(14, -1, -1):
                tr = ipref + jnp.int32(1 << b)
                ipref = jnp.where(csel(tr) < need, tr, ipref)
            return ipref

        cutoff = lax.cond(jnp.any(cl > _K), tie_cutoff,
                          lambda: _splat(_N - 1))

        def out_step(j, carry):
            base = j * 64
            for g in range(4):
                sl = pl.ds(base + g * _L, _L)
                v = row_v[sl]
                k = key_v[sl]
                idx = lax.iota(jnp.int32, _L) + (base + g * _L)
                mask = (k > t) | ((k == t) & (idx <= cutoff))
                row_v[sl] = jnp.where(mask, jnp.maximum(v, 0.0), 0.0)
            return carry

        lax.fori_loop(0, _N // 64, out_step, 0)
        pltpu.sync_copy(row_v, out_hbm.at[row])


def kernel(x):
    mesh = plsc.VectorSubcoreMesh(core_axis_name="c", subcore_axis_name="s")
    f = functools.partial(
        pl.kernel,
        mesh=mesh,
        compiler_params=pltpu.CompilerParams(needs_layout_passes=False),
        out_type=jax.ShapeDtypeStruct((_ROWS, _N), jnp.float32),
        scratch_types=[
            pltpu.VMEM((_N,), jnp.float32),
            pltpu.VMEM((_N,), jnp.int32),
            pltpu.SemaphoreType.DMA,
        ],
    )(_sc_body)
    return f(x)


# speedup vs baseline: 1.9881x; 1.9881x over previous
import functools
import jax
import jax.numpy as jnp
from jax import lax
from jax.experimental import pallas as pl
from jax.experimental.pallas import tpu as pltpu
from jax.experimental.pallas import tpu_sc as plsc

_K = 64
_N = 32768
_ROWS = 128
_TC_ROWS = 96
_SC_ROWS = _ROWS - _TC_ROWS
_TC_RB = 32
_LANES = 16
_INT_MIN32 = -2147483648


def _bitval(b):
    return jnp.int32(_INT_MIN32) if b == 31 else jnp.int32(1 << b)


def _tc_body(x_ref, o_ref):
    x = x_ref[...]
    r, n = x.shape

    raw = lax.bitcast_convert_type(x, jnp.int32)
    ikey = jnp.where(raw >= 0, raw, raw ^ jnp.int32(0x7FFFFFFF))

    def _lane_sum(y):
        part = jnp.sum(y.reshape(r, y.shape[1] // 128, 128), axis=1)
        return jnp.sum(part, axis=1, keepdims=True)

    def count_ge(t):
        return _lane_sum((ikey >= t).astype(jnp.int32))

    m = jnp.max(ikey.reshape(r, n // 512, 512), axis=1)
    u = jnp.max(m, axis=1, keepdims=True)

    lp = jnp.full((r, 1), _INT_MIN32, jnp.int32)
    for b in range(31, 11, -1):
        tr = lp ^ _bitval(b)
        c = _lane_sum((m >= tr).astype(jnp.int32))
        lp = jnp.where(c >= _K, tr, lp)

    lo0 = lp
    cl0 = count_ge(lo0)
    hi0 = u + 1
    ch0 = jnp.zeros((r, 1), jnp.int32)

    def active(lo, hi, cl):
        return (cl > _K) & (hi - 1 > lo)

    def loop_cond(state):
        it, lo, hi, cl, ch = state
        return jnp.any(active(lo, hi, cl))

    def loop_body(state):
        it, lo, hi, cl, ch = state
        act = active(lo, hi, cl)
        wf = hi.astype(jnp.float32) - lo.astype(jnp.float32)
        frac = (cl - _K).astype(jnp.float32) / jnp.maximum(
            (cl - ch).astype(jnp.float32), 1.0)
        stepf = jnp.clip(wf * frac, 1.0, jnp.maximum(wf - 1.0, 1.0))
        mid_i = lo + stepf.astype(jnp.int32)
        mid_b = (lo & hi) + ((lo ^ hi) >> 1)
        mid = jnp.where(it % 2 == 0, mid_i, mid_b)
        mid = jnp.maximum(lo + 1, jnp.minimum(mid, hi - 1))
        c = count_ge(mid)
        up = c >= _K
        lo = jnp.where(act & up, mid, lo)
        cl = jnp.where(act & up, c, cl)
        hi = jnp.where(act & ~up, mid, hi)
        ch = jnp.where(act & ~up, c, ch)
        return it + 1, lo, hi, cl, ch

    _, lo, hi, cl, ch = lax.while_loop(
        loop_cond, loop_body, (jnp.int32(0), lo0, hi0, cl0, ch0))

    t = lo
    ties_any = jnp.any(cl > _K)

    @pl.when(jnp.logical_not(ties_any))
    def _no_ties():
        o_ref[...] = jnp.where(ikey >= t, jnp.maximum(x, 0.0), 0.0)

    @pl.when(ties_any)
    def _with_ties():
        gt = ikey > t
        eq = ikey == t
        cnt_gt = _lane_sum(gt.astype(jnp.int32))
        need = _K - cnt_gt
        idx = lax.broadcasted_iota(jnp.int32, x.shape, 1)
        ipref = jnp.zeros((r, 1), dtype=jnp.int32)
        for b in range(14, -1, -1):
            tr = ipref + jnp.int32(1 << b)
            c = _lane_sum((eq & (idx < tr)).astype(jnp.int32))
            ipref = jnp.where(c < need, tr, ipref)
        mask = gt | (eq & (idx <= ipref))
        o_ref[...] = jnp.where(mask, jnp.maximum(x, 0.0), 0.0)


def _tc_kernel(x):
    return pl.pallas_call(
        _tc_body,
        grid=(_TC_ROWS // _TC_RB,),
        in_specs=[pl.BlockSpec((_TC_RB, _N), lambda i: (i, 0))],
        out_specs=pl.BlockSpec((_TC_RB, _N), lambda i: (i, 0)),
        out_shape=jax.ShapeDtypeStruct((_TC_ROWS, _N), x.dtype),
    )(x)


def _splat(s):
    return jnp.full((_LANES,), s, jnp.int32)


def _popcount(mask):
    return plsc.all_reduce_population_count(mask)


def _lane_total(acc, nbits=12):
    tot = _splat(0)
    for b in range(nbits):
        pc = _popcount(((acc >> b) & 1) == 1)
        tot = tot + (pc << b)
    return tot


def _vreg_kth(vec, k):
    prefix = _splat(_INT_MIN32)
    for b in range(31, -1, -1):
        tr = prefix ^ _bitval(b)
        c = _popcount(vec >= tr)
        prefix = jnp.where(c >= k, tr, prefix)
    return prefix


def _sc_body(x_hbm, out_hbm, row_v, key_v, sem):
    wid = lax.axis_index("s") * 2 + lax.axis_index("c")
    row = _TC_ROWS + wid
    pltpu.sync_copy(x_hbm.at[row], row_v)

    def mx_step(j, accs):
        base = j * 64
        out = []
        for g in range(4):
            sl = pl.ds(base + g * _LANES, _LANES)
            raw = lax.bitcast_convert_type(row_v[sl], jnp.int32)
            k = jnp.where(raw >= 0, raw, raw ^ jnp.int32(0x7FFFFFFF))
            key_v[sl] = k
            out.append(jnp.maximum(accs[g], k))
        return tuple(out)

    a = lax.fori_loop(0, _N // 64, mx_step,
                      tuple(_splat(_INT_MIN32) for _ in range(4)))
    mall = jnp.maximum(jnp.maximum(a[0], a[1]), jnp.maximum(a[2], a[3]))
    mmin = jnp.minimum(jnp.minimum(a[0], a[1]), jnp.minimum(a[2], a[3]))
    u = _vreg_kth(mall, 1)
    lo0 = _vreg_kth(mmin, _LANES)

    def count_ge(tsplat):
        def step(j, acc):
            base = j * 64
            s = acc
            for g in range(4):
                k = key_v[pl.ds(base + g * _LANES, _LANES)]
                s = s + (k >= tsplat).astype(jnp.int32)
            return s

        acc = lax.fori_loop(0, _N // 64, step, _splat(0))
        return _lane_total(acc)

    cl0 = count_ge(lo0)
    hi0 = u + 1
    ch0 = _splat(0)

    def active(lo, hi, cl):
        return (cl > _K) & (hi - 1 > lo)

    def loop_cond(s):
        it, lo, hi, cl, ch = s
        return jnp.any(active(lo, hi, cl))

    def loop_body(s):
        it, lo, hi, cl, ch = s
        wf = hi.astype(jnp.float32) - lo.astype(jnp.float32)
        frac = (cl - _K).astype(jnp.float32) / jnp.maximum(
            (cl - ch).astype(jnp.float32), 1.0)
        stepf = jnp.clip(wf * frac, 1.0, jnp.maximum(wf - 1.0, 1.0))
        mid_i = lo + stepf.astype(jnp.int32)
        mid_b = (lo & hi) + ((lo ^ hi) >> 1)
        mid = jnp.where(it % 2 == 0, mid_i, mid_b)
        mid = jnp.maximum(lo + 1, jnp.minimum(mid, hi - 1))
        c = count_ge(mid)
        up = c >= _K
        lo = jnp.where(up, mid, lo)
        cl = jnp.where(up, c, cl)
        hi = jnp.where(up, hi, mid)
        ch = jnp.where(up, ch, c)
        return it + 1, lo, hi, cl, ch

    _, lo, hi, cl, ch = lax.while_loop(
        loop_cond, loop_body, (jnp.int32(0), lo0, hi0, cl0, ch0))
    t = lo

    def tie_cutoff():
        cnt_gt = count_ge(t + 1)
        need = _K - cnt_gt

        def csel(trs):
            def step(j, acc):
                base = j * 64
                s = acc
                for g in range(4):
                    off = base + g * _LANES
                    k = key_v[pl.ds(off, _LANES)]
                    idx = lax.iota(jnp.int32, _LANES) + off
                    s = s + ((k == t) & (idx < trs)).astype(jnp.int32)
                return s

            acc = lax.fori_loop(0, _N // 64, step, _splat(0))
            return _lane_total(acc)

        ipref = _splat(0)
        for b in range(14, -1, -1):
            tr = ipref + jnp.int32(1 << b)
            ipref = jnp.where(csel(tr) < need, tr, ipref)
        return ipref

    cutoff = lax.cond(jnp.any(cl > _K), tie_cutoff, lambda: _splat(_N - 1))

    def out_step(j, carry):
        base = j * 64
        for g in range(4):
            sl = pl.ds(base + g * _LANES, _LANES)
            v = row_v[sl]
            k = key_v[sl]
            idx = lax.iota(jnp.int32, _LANES) + (base + g * _LANES)
            mask = (k > t) | ((k == t) & (idx <= cutoff))
            row_v[sl] = jnp.where(mask, jnp.maximum(v, 0.0), 0.0)
        return carry

    lax.fori_loop(0, _N // 64, out_step, 0)
    pltpu.sync_copy(row_v, out_hbm.at[row - _TC_ROWS])


def _sc_kernel(x):
    mesh = plsc.VectorSubcoreMesh(core_axis_name="c", subcore_axis_name="s")
    f = functools.partial(
        pl.kernel,
        mesh=mesh,
        compiler_params=pltpu.CompilerParams(needs_layout_passes=False),
        out_type=jax.ShapeDtypeStruct((_SC_ROWS, _N), jnp.float32),
        scratch_types=[
            pltpu.VMEM((_N,), jnp.float32),
            pltpu.VMEM((_N,), jnp.int32),
            pltpu.SemaphoreType.DMA,
        ],
    )(_sc_body)
    return f(x)


def kernel(x):
    tc_out = _tc_kernel(x)
    sc_out = _sc_kernel(x)
    return jnp.concatenate([tc_out, sc_out], axis=0)


# speedup vs baseline: 2.1681x; 1.0905x over previous
import functools
import jax
import jax.numpy as jnp
from jax import lax
from jax.experimental import pallas as pl
from jax.experimental.pallas import tpu as pltpu
from jax.experimental.pallas import tpu_sc as plsc

_K = 64
_N = 32768
_ROWS = 128
_TC_ROWS = 96
_SC_ROWS = _ROWS - _TC_ROWS
_TC_RB = 32
_LANES = 16
_INT_MIN32 = -2147483648


def _bitval(b):
    return jnp.int32(_INT_MIN32) if b == 31 else jnp.int32(1 << b)


def _tc_body(x_ref, o_ref):
    x = x_ref[...]
    r, n = x.shape

    raw = lax.bitcast_convert_type(x, jnp.int32)
    ikey = jnp.where(raw >= 0, raw, raw ^ jnp.int32(0x7FFFFFFF))

    def _lane_sum(y):
        part = jnp.sum(y.reshape(r, y.shape[1] // 128, 128), axis=1)
        return jnp.sum(part, axis=1, keepdims=True)

    def count_ge(t):
        return _lane_sum((ikey >= t).astype(jnp.int32))

    m = jnp.max(ikey.reshape(r, n // 512, 512), axis=1)
    u = jnp.max(m, axis=1, keepdims=True)

    lp = jnp.full((r, 1), _INT_MIN32, jnp.int32)
    for b in range(31, 11, -1):
        tr = lp ^ _bitval(b)
        c = _lane_sum((m >= tr).astype(jnp.int32))
        lp = jnp.where(c >= _K, tr, lp)

    lo0 = lp
    cl0 = count_ge(lo0)
    hi0 = u + 1
    ch0 = jnp.zeros((r, 1), jnp.int32)

    def active(lo, hi, cl):
        return (cl > _K) & (hi - 1 > lo)

    def loop_cond(state):
        it, lo, hi, cl, ch = state
        return jnp.any(active(lo, hi, cl))

    def loop_body(state):
        it, lo, hi, cl, ch = state
        act = active(lo, hi, cl)
        wf = hi.astype(jnp.float32) - lo.astype(jnp.float32)
        frac = (cl - _K).astype(jnp.float32) / jnp.maximum(
            (cl - ch).astype(jnp.float32), 1.0)
        stepf = jnp.clip(wf * frac, 1.0, jnp.maximum(wf - 1.0, 1.0))
        mid_i = lo + stepf.astype(jnp.int32)
        mid_b = (lo & hi) + ((lo ^ hi) >> 1)
        mid = jnp.where(it % 2 == 0, mid_i, mid_b)
        mid = jnp.maximum(lo + 1, jnp.minimum(mid, hi - 1))
        c = count_ge(mid)
        up = c >= _K
        lo = jnp.where(act & up, mid, lo)
        cl = jnp.where(act & up, c, cl)
        hi = jnp.where(act & ~up, mid, hi)
        ch = jnp.where(act & ~up, c, ch)
        return it + 1, lo, hi, cl, ch

    _, lo, hi, cl, ch = lax.while_loop(
        loop_cond, loop_body, (jnp.int32(0), lo0, hi0, cl0, ch0))

    t = lo
    ties_any = jnp.any(cl > _K)

    @pl.when(jnp.logical_not(ties_any))
    def _no_ties():
        o_ref[...] = jnp.where(ikey >= t, jnp.maximum(x, 0.0), 0.0)

    @pl.when(ties_any)
    def _with_ties():
        gt = ikey > t
        eq = ikey == t
        cnt_gt = _lane_sum(gt.astype(jnp.int32))
        need = _K - cnt_gt
        idx = lax.broadcasted_iota(jnp.int32, x.shape, 1)
        ipref = jnp.zeros((r, 1), dtype=jnp.int32)
        for b in range(14, -1, -1):
            tr = ipref + jnp.int32(1 << b)
            c = _lane_sum((eq & (idx < tr)).astype(jnp.int32))
            ipref = jnp.where(c < need, tr, ipref)
        mask = gt | (eq & (idx <= ipref))
        o_ref[...] = jnp.where(mask, jnp.maximum(x, 0.0), 0.0)


def _tc_kernel(x):
    return pl.pallas_call(
        _tc_body,
        grid=(_TC_ROWS // _TC_RB,),
        in_specs=[pl.BlockSpec((_TC_RB, _N), lambda i: (i, 0))],
        out_specs=pl.BlockSpec((_TC_RB, _N), lambda i: (i, 0)),
        out_shape=jax.ShapeDtypeStruct((_ROWS, _N), x.dtype),
    )(x)


def _splat(s):
    return jnp.full((_LANES,), s, jnp.int32)


def _popcount(mask):
    return plsc.all_reduce_population_count(mask)


def _lane_total(acc, nbits=12):
    tot = _splat(0)
    for b in range(nbits):
        pc = _popcount(((acc >> b) & 1) == 1)
        tot = tot + (pc << b)
    return tot


def _vreg_kth(vec, k):
    prefix = _splat(_INT_MIN32)
    for b in range(31, -1, -1):
        tr = prefix ^ _bitval(b)
        c = _popcount(vec >= tr)
        prefix = jnp.where(c >= k, tr, prefix)
    return prefix


def _sc_body(x_hbm, out_hbm, row_v, key_v, sem):
    wid = lax.axis_index("s") * 2 + lax.axis_index("c")
    row = _TC_ROWS + wid
    pltpu.sync_copy(x_hbm.at[row], row_v)

    def mx_step(j, accs):
        base = j * 64
        out = []
        for g in range(4):
            sl = pl.ds(base + g * _LANES, _LANES)
            raw = lax.bitcast_convert_type(row_v[sl], jnp.int32)
            k = jnp.where(raw >= 0, raw, raw ^ jnp.int32(0x7FFFFFFF))
            key_v[sl] = k
            out.append(jnp.maximum(accs[g], k))
        return tuple(out)

    a = lax.fori_loop(0, _N // 64, mx_step,
                      tuple(_splat(_INT_MIN32) for _ in range(4)))
    mall = jnp.maximum(jnp.maximum(a[0], a[1]), jnp.maximum(a[2], a[3]))
    mmin = jnp.minimum(jnp.minimum(a[0], a[1]), jnp.minimum(a[2], a[3]))
    u = _vreg_kth(mall, 1)
    lo0 = _vreg_kth(mmin, _LANES)

    def count_ge(tsplat):
        def step(j, acc):
            base = j * 64
            s = acc
            for g in range(4):
                k = key_v[pl.ds(base + g * _LANES, _LANES)]
                s = s + (k >= tsplat).astype(jnp.int32)
            return s

        acc = lax.fori_loop(0, _N // 64, step, _splat(0))
        return _lane_total(acc)

    cl0 = count_ge(lo0)
    hi0 = u + 1
    ch0 = _splat(0)

    def active(lo, hi, cl):
        return (cl > _K) & (hi - 1 > lo)

    def loop_cond(s):
        it, lo, hi, cl, ch = s
        return jnp.any(active(lo, hi, cl))

    def loop_body(s):
        it, lo, hi, cl, ch = s
        wf = hi.astype(jnp.float32) - lo.astype(jnp.float32)
        frac = (cl - _K).astype(jnp.float32) / jnp.maximum(
            (cl - ch).astype(jnp.float32), 1.0)
        stepf = jnp.clip(wf * frac, 1.0, jnp.maximum(wf - 1.0, 1.0))
        mid_i = lo + stepf.astype(jnp.int32)
        mid_b = (lo & hi) + ((lo ^ hi) >> 1)
        mid = jnp.where(it % 2 == 0, mid_i, mid_b)
        mid = jnp.maximum(lo + 1, jnp.minimum(mid, hi - 1))
        c = count_ge(mid)
        up = c >= _K
        lo = jnp.where(up, mid, lo)
        cl = jnp.where(up, c, cl)
        hi = jnp.where(up, hi, mid)
        ch = jnp.where(up, ch, c)
        return it + 1, lo, hi, cl, ch

    _, lo, hi, cl, ch = lax.while_loop(
        loop_cond, loop_body, (jnp.int32(0), lo0, hi0, cl0, ch0))
    t = lo

    def tie_cutoff():
        cnt_gt = count_ge(t + 1)
        need = _K - cnt_gt

        def csel(trs):
            def step(j, acc):
                base = j * 64
                s = acc
                for g in range(4):
                    off = base + g * _LANES
                    k = key_v[pl.ds(off, _LANES)]
                    idx = lax.iota(jnp.int32, _LANES) + off
                    s = s + ((k == t) & (idx < trs)).astype(jnp.int32)
                return s

            acc = lax.fori_loop(0, _N // 64, step, _splat(0))
            return _lane_total(acc)

        ipref = _splat(0)
        for b in range(14, -1, -1):
            tr = ipref + jnp.int32(1 << b)
            ipref = jnp.where(csel(tr) < need, tr, ipref)
        return ipref

    cutoff = lax.cond(jnp.any(cl > _K), tie_cutoff, lambda: _splat(_N - 1))

    def out_step(j, carry):
        base = j * 64
        for g in range(4):
            sl = pl.ds(base + g * _LANES, _LANES)
            v = row_v[sl]
            k = key_v[sl]
            idx = lax.iota(jnp.int32, _LANES) + (base + g * _LANES)
            mask = (k > t) | ((k == t) & (idx <= cutoff))
            row_v[sl] = jnp.where(mask, jnp.maximum(v, 0.0), 0.0)
        return carry

    lax.fori_loop(0, _N // 64, out_step, 0)
    pltpu.sync_copy(row_v, out_hbm.at[row - _TC_ROWS])


def _sc_kernel(x):
    mesh = plsc.VectorSubcoreMesh(core_axis_name="c", subcore_axis_name="s")
    f = functools.partial(
        pl.kernel,
        mesh=mesh,
        compiler_params=pltpu.CompilerParams(needs_layout_passes=False),
        out_type=jax.ShapeDtypeStruct((_SC_ROWS, _N), jnp.float32),
        scratch_types=[
            pltpu.VMEM((_N,), jnp.float32),
            pltpu.VMEM((_N,), jnp.int32),
            pltpu.SemaphoreType.DMA,
        ],
    )(_sc_body)
    return f(x)


def kernel(x):
    tc_out = _tc_kernel(x)
    sc_out = _sc_kernel(x)
    return lax.dynamic_update_slice(tc_out, sc_out, (_TC_ROWS, 0))
